# Initial kernel scaffold; baseline (speedup 1.0000x reference)
import jax
import jax.numpy as jnp
from jax.experimental import pallas as pl


def kernel(x, edge_index, batch, W1l, W1r, b1, W2l, W2r, b2, p1w, p2w, fc1W, fc1b, fc2W, fc2b):
    raise NotImplementedError("write your pallas kernel here")


# speedup vs baseline: 10.5551x; 10.5551x over previous
import functools

import jax
import jax.numpy as jnp
from jax import lax
from jax.experimental import pallas as pl
from jax.experimental.pallas import tpu as pltpu
from jax.experimental.pallas import tpu_sc as plsc

_HID = 128
_CLS = 10
_G = 64
_N = 10000
_NPAD = 10240
_E = 320000
_CHUNK = 128
_CPT = 79
_EPT = _CPT * _CHUNK
_EPAD = 32 * _EPT
_ROWS_PER_TILE = _NPAD // 16
_BIS_ITERS = 44


def _sc_scatter_body(feat_hbm, w_hbm, src_hbm, dst_hbm, out_hbm, outc_hbm,
                     sidx, didx, rows, wvec, cnt, acc, sem):
    c = lax.axis_index("c")
    s = lax.axis_index("s")
    wid = c * 16 + s

    pltpu.sync_copy(w_hbm, wvec)
    def zc(i, _):
        cnt[pl.ds(i * 16, 16)] = jnp.zeros((16,), jnp.float32)
        return 0
    lax.fori_loop(0, _NPAD // 16, zc, 0)

    def zr(r, _):
        def zrc(j, _):
            rows[r, pl.ds(j * 16, 16)] = jnp.zeros((16,), jnp.float32)
            return 0
        return lax.fori_loop(0, _HID // 16, zrc, 0)
    lax.fori_loop(0, _CHUNK, zr, 0)
    base_row = s * _ROWS_PER_TILE
    def zacc(j, _):
        pltpu.sync_copy(rows, acc.at[pl.ds(base_row + j * _CHUNK, _CHUNK)])
        return 0
    lax.fori_loop(0, _ROWS_PER_TILE // _CHUNK, zacc, 0)
    plsc.subcore_barrier()

    def body(i, _):
        off = wid * _EPT + i * _CHUNK
        pltpu.sync_copy(src_hbm.at[pl.ds(off, _CHUNK)], sidx)
        pltpu.sync_copy(dst_hbm.at[pl.ds(off, _CHUNK)], didx)
        pltpu.async_copy(feat_hbm.at[sidx], rows, sem).wait()
        pltpu.sync_copy(rows, acc.at[didx], add=True)
        for j in range(_CHUNK // 16):
            sj = sidx[pl.ds(j * 16, 16)]
            dj = didx[pl.ds(j * 16, 16)]
            wj = plsc.load_gather(wvec, [sj])
            plsc.addupdate_scatter(cnt, [dj], wj)
        return 0
    lax.fori_loop(0, _CPT, body, 0)

    plsc.subcore_barrier()
    pltpu.sync_copy(acc.at[pl.ds(base_row, _ROWS_PER_TILE)],
                    out_hbm.at[c, pl.ds(base_row, _ROWS_PER_TILE)])
    pltpu.sync_copy(cnt, outc_hbm.at[wid])


def _sc_scatter(feat, w, src, dst):
    mesh = plsc.VectorSubcoreMesh(core_axis_name="c", subcore_axis_name="s")
    f = pl.kernel(
        _sc_scatter_body,
        out_type=(
            jax.ShapeDtypeStruct((2, _NPAD, _HID), jnp.float32),
            jax.ShapeDtypeStruct((32, _NPAD), jnp.float32),
        ),
        mesh=mesh,
        scratch_types=[
            pltpu.VMEM((_CHUNK,), jnp.int32),
            pltpu.VMEM((_CHUNK,), jnp.int32),
            pltpu.VMEM((_CHUNK, _HID), jnp.float32),
            pltpu.VMEM((_NPAD,), jnp.float32),
            pltpu.VMEM((_NPAD,), jnp.float32),
            pltpu.VMEM_SHARED((_NPAD, _HID), jnp.float32),
            pltpu.SemaphoreType.DMA,
        ],
        compiler_params=pltpu.CompilerParams(needs_layout_passes=False),
    )
    return f(feat, w, src, dst)


def _h_body(aggp_ref, cntp_ref, xin_ref, wl_ref, wr_ref, b_ref, h_ref):
    agg = aggp_ref[0] + aggp_ref[1]
    ones32 = jnp.ones((32, 1), jnp.float32)
    cnt = lax.dot_general(cntp_ref[...], ones32, (((0,), (0,)), ((), ())))
    mean = agg / jnp.maximum(cnt, 1.0)
    h = (jnp.dot(mean, wl_ref[...], preferred_element_type=jnp.float32)
         + jnp.dot(xin_ref[...], wr_ref[...], preferred_element_type=jnp.float32)
         + b_ref[...])
    h_ref[...] = jnp.maximum(h, 0.0)


def _sage_h(aggp, cntp, xin, wl, wr, b):
    blk = 1024
    grid = _NPAD // blk
    return pl.pallas_call(
        _h_body,
        grid=(grid,),
        in_specs=[
            pl.BlockSpec((2, blk, _HID), lambda i: (0, i, 0)),
            pl.BlockSpec((32, blk), lambda i: (0, i)),
            pl.BlockSpec((blk, _HID), lambda i: (i, 0)),
            pl.BlockSpec((_HID, _HID), lambda i: (0, 0)),
            pl.BlockSpec((_HID, _HID), lambda i: (0, 0)),
            pl.BlockSpec((1, _HID), lambda i: (0, 0)),
        ],
        out_specs=pl.BlockSpec((blk, _HID), lambda i: (i, 0)),
        out_shape=jax.ShapeDtypeStruct((_NPAD, _HID), jnp.float32),
    )(aggp, cntp, xin, wl, wr, b)


def _pool_core(h, batch, pw, keep_prev, base):
    gids = lax.broadcasted_iota(jnp.int32, (1, _G), 1)
    bf = (batch == gids).astype(jnp.float32)
    norm = jnp.sqrt(jnp.sum(pw * pw))
    raw = lax.dot_general(h, pw, (((1,), (1,)), ((), ()))) / norm
    s = jnp.tanh(raw)
    s_eff = jnp.where(keep_prev, s, -2.0)
    k = jnp.floor((4.0 * base + 4.0) / 5.0)

    lo0 = jnp.full((_G, 1), -3.0, jnp.float32)
    hi0 = jnp.full((_G, 1), 1.0, jnp.float32)

    def bis(_, carry):
        lo, hi = carry
        mid = 0.5 * (lo + hi)
        thr = jnp.dot(bf, mid, preferred_element_type=jnp.float32)
        ind = (s_eff >= thr).astype(jnp.float32)
        cntg = lax.dot_general(bf, ind, (((0,), (0,)), ((), ())))
        pred = cntg >= k
        return jnp.where(pred, mid, lo), jnp.where(pred, hi, mid)

    lo, hi = lax.fori_loop(0, _BIS_ITERS, bis, (lo0, hi0))
    thr = jnp.dot(bf, lo, preferred_element_type=jnp.float32)
    keep = s_eff >= thr
    keepf = keep.astype(jnp.float32)
    xp = h * (s * keepf)
    gap = lax.dot_general(bf, xp, (((0,), (0,)), ((), ()))) / jnp.maximum(k, 1.0)
    return xp, keep, keepf, gap, k


def _write_gmp(gmp_ref, xp, batch, keep, k):
    def body(g, _):
        m = (batch == g) & keep
        col = jnp.max(jnp.where(m, xp, -1e30), axis=0, keepdims=True)
        gmp_ref[pl.ds(g, 1), :] = col
        return 0
    lax.fori_loop(0, _G, body, 0)
    gmp_ref[...] = jnp.where(k > 0, gmp_ref[...], 0.0)


def _pool1_body(h_ref, batch_ref, pw_ref, xp_ref, keep_ref, gap_ref, gmp_ref,
                ks_ref):
    h = h_ref[...]
    batch = batch_ref[...]
    valid = lax.broadcasted_iota(jnp.int32, (_NPAD, 1), 0) < _N
    gids = lax.broadcasted_iota(jnp.int32, (1, _G), 1)
    bf = (batch == gids).astype(jnp.float32)
    sizes = jnp.sum(bf, axis=0)[:, None]
    xp, keep, keepf, gap, k = _pool_core(h, batch, pw_ref[...], valid, sizes)
    xp_ref[...] = xp
    keep_ref[...] = keepf
    gap_ref[...] = gap
    ks_ref[...] = k
    _write_gmp(gmp_ref, xp, batch, keep, k)


def _pool2_body(h_ref, batch_ref, pw_ref, keep1_ref, ks1_ref, gap1_ref,
                gmp1_ref, fc1w_ref, fc1b_ref, fc2w_ref, fc2b_ref, out_ref,
                gmp_scr):
    h = h_ref[...]
    batch = batch_ref[...]
    valid = lax.broadcasted_iota(jnp.int32, (_NPAD, 1), 0) < _N
    keep1 = (keep1_ref[...] > 0.5) & valid
    xp, keep, keepf, gap2, k2 = _pool_core(h, batch, pw_ref[...], keep1,
                                           ks1_ref[...])
    _write_gmp(gmp_scr, xp, batch, keep, k2)
    zg = gap1_ref[...] + gap2
    zm = gmp1_ref[...] + gmp_scr[...]
    z = jnp.concatenate([zg, zm], axis=1)
    z = jnp.maximum(jnp.dot(z, fc1w_ref[...], preferred_element_type=jnp.float32)
                    + fc1b_ref[...], 0.0)
    out_ref[...] = (jnp.dot(z, fc2w_ref[...], preferred_element_type=jnp.float32)
                    + fc2b_ref[...])


def _pool1(h, batchc, pw):
    return pl.pallas_call(
        _pool1_body,
        out_shape=(
            jax.ShapeDtypeStruct((_NPAD, _HID), jnp.float32),
            jax.ShapeDtypeStruct((_NPAD, 1), jnp.float32),
            jax.ShapeDtypeStruct((_G, _HID), jnp.float32),
            jax.ShapeDtypeStruct((_G, _HID), jnp.float32),
            jax.ShapeDtypeStruct((_G, 1), jnp.float32),
        ),
    )(h, batchc, pw)


def _pool2_head(h, batchc, pw, keep1, ks1, gap1, gmp1, fc1w, fc1b, fc2wp, fc2bp):
    return pl.pallas_call(
        _pool2_body,
        out_shape=jax.ShapeDtypeStruct((_G, _HID), jnp.float32),
        scratch_shapes=[pltpu.VMEM((_G, _HID), jnp.float32)],
    )(h, batchc, pw, keep1, ks1, gap1, gmp1, fc1w, fc1b, fc2wp, fc2bp)


def kernel(x, edge_index, batch, W1l, W1r, b1, W2l, W2r, b2, p1w, p2w,
           fc1W, fc1b, fc2W, fc2b):
    xpad = jnp.pad(x, ((0, _NPAD - _N), (0, 0)))
    w1 = jnp.pad(jnp.ones((_N,), jnp.float32), (0, _NPAD - _N))
    src = jnp.pad(edge_index[0], (0, _EPAD - _E), constant_values=_NPAD - 1)
    dst = jnp.pad(edge_index[1], (0, _EPAD - _E), constant_values=_NPAD - 1)
    batchc = jnp.pad(batch, (0, _NPAD - _N), constant_values=_G)[:, None]
    b1r = b1[None, :]
    b2r = b2[None, :]
    p1r = p1w[None, :]
    p2r = p2w[None, :]
    fc1br = fc1b[None, :]
    fc2wp = jnp.pad(fc2W, ((0, 0), (0, _HID - _CLS)))
    fc2bp = jnp.pad(fc2b, (0, _HID - _CLS))[None, :]

    aggp1, cntp1 = _sc_scatter(xpad, w1, src, dst)
    h1 = _sage_h(aggp1, cntp1, xpad, W1l, W1r, b1r)
    xp1, keep1, gap1, gmp1, ks1 = _pool1(h1, batchc, p1r)

    aggp2, cntp2 = _sc_scatter(xp1, jnp.reshape(keep1, (_NPAD,)), src, dst)
    h2 = _sage_h(aggp2, cntp2, xp1, W2l, W2r, b2r)
    out128 = _pool2_head(h2, batchc, p2r, keep1, ks1, gap1, gmp1,
                         fc1W, fc1br, fc2wp, fc2bp)
    return out128[:, :_CLS]
